# Initial kernel scaffold; baseline (speedup 1.0000x reference)
import jax
import jax.numpy as jnp
from jax.experimental import pallas as pl


def kernel(x, nb):
    raise NotImplementedError("write your pallas kernel here")


# speedup vs baseline: 2.5020x; 2.5020x over previous
import jax
import jax.numpy as jnp
from jax import lax
from jax.experimental import pallas as pl
from jax.experimental.pallas import tpu as pltpu
from jax.experimental.pallas import tpu_sc as plsc

E = 320000
C = 128
T = 160000
NC = 2
NS = 16
W = NC * NS
PW = E // W
SEG = T // W
SEGP = SEG + 8
NB1 = 4096
NB2 = 4096
NB3 = 256
BLK = 512

_mesh = lambda: plsc.VectorSubcoreMesh(core_axis_name="c", subcore_axis_name="s")


def _wid():
    return lax.axis_index("s") * NC + lax.axis_index("c")


def _iota16():
    return lax.iota(jnp.int32, 16)


def _splat(vref, i, tz):
    return plsc.load_gather(vref, [jnp.zeros((16,), jnp.int32) + tz + i])


def _norms(x):
    def body(xr, out):
        xb = xr[...]
        y = xb * xb
        acc = y[:, 0:8]
        for k in range(1, 16):
            acc = acc + y[:, 8 * k:8 * k + 8]
        w = 8
        while w > 1:
            w //= 2
            acc = acc[:, :w] + acc[:, w:2 * w]
        out[...] = jnp.sqrt(acc[:, 0])

    return pl.pallas_call(
        body,
        grid=(E // BLK,),
        in_specs=[pl.BlockSpec((BLK, C), lambda i: (i, 0))],
        out_specs=pl.BlockSpec((BLK,), lambda i: (i,)),
        out_shape=jax.ShapeDtypeStruct((E,), jnp.float32),
    )(x)


def _sc_hist(norms, sel, level):
    nb = (NB1, NB2, NB3)[level]
    nsel = 0 if level == 0 else 128

    def body(*refs):
        if level == 0:
            norms_hbm, h_hbm, nch, hist, sem = refs
            selv = None
        else:
            norms_hbm, sel_hbm, h_hbm, nch, hist, selv, sem = refs
        wid = _wid()
        base = wid * PW
        pltpu.sync_copy(norms_hbm.at[pl.ds(base, PW)], nch)
        if level > 0:
            pltpu.sync_copy(sel_hbm, selv)
            key_s = _splat(selv, 1, 0)
        iota = _iota16()
        ones = jnp.ones((16,), jnp.int32)

        def clr(i, _):
            hist[pl.ds(i * 16, 16)] = jnp.zeros((16,), jnp.int32)
            return 0

        lax.fori_loop(0, nb, clr, 0)

        def acc(j, _):
            u = nch[pl.ds(j * 16, 16)]
            if level == 0:
                b = lax.shift_right_logical(u, 20)
                m = iota < 16
            elif level == 1:
                b = lax.shift_right_logical(u, 8) & 0xFFF
                m = lax.shift_right_logical(u, 20) == key_s
            else:
                b = u & 0xFF
                m = lax.shift_right_logical(u, 8) == key_s
            idx = iota * nb + b
            plsc.addupdate_scatter(hist, [idx], ones, mask=m)
            return 0

        lax.fori_loop(0, PW // 16, acc, 0)
        pltpu.sync_copy(hist, h_hbm.at[wid])

    scratch = [pltpu.VMEM((PW,), jnp.int32), pltpu.VMEM((16 * nb,), jnp.int32)]
    if level > 0:
        scratch.append(pltpu.VMEM((nsel,), jnp.int32))
    scratch.append(pltpu.SemaphoreType.DMA)

    kfn = pl.kernel(
        body,
        out_type=jax.ShapeDtypeStruct((W, 16 * nb), jnp.int32),
        mesh=_mesh(),
        compiler_params=pltpu.CompilerParams(needs_layout_passes=False),
        scratch_types=scratch,
    )
    return kfn(norms) if level == 0 else kfn(norms, sel)


def _suffix_pick(counts, kk):
    nb = counts.shape[0]
    rows, cols = nb // 128, 128
    M = counts.reshape(rows, cols)
    rs = jnp.sum(M, axis=1)
    rj = lax.broadcasted_iota(jnp.int32, (rows, rows), 1)
    ri = lax.broadcasted_iota(jnp.int32, (rows, rows), 0)
    S = jnp.sum(jnp.where(rj >= ri, rs[None, :], 0), axis=1)
    rstar = jnp.sum((S >= kk).astype(jnp.int32)) - 1
    riota = lax.broadcasted_iota(jnp.int32, (rows, cols), 0)
    row = jnp.sum(jnp.where(riota == rstar, M, 0), axis=0)
    ir = lax.iota(jnp.int32, rows)
    sx_star = jnp.sum(jnp.where(ir == rstar, S - rs, 0))
    cj = lax.broadcasted_iota(jnp.int32, (cols, cols), 1)
    ci = lax.broadcasted_iota(jnp.int32, (cols, cols), 0)
    csfx = jnp.sum(jnp.where(cj >= ci, row[None, :], 0), axis=1) + sx_star
    cstar = jnp.sum((csfx >= kk).astype(jnp.int32)) - 1
    b = rstar * cols + cstar
    ic = lax.iota(jnp.int32, cols)
    cnt_above = jnp.sum(jnp.where(ic == cstar, csfx - row, 0))
    return b, cnt_above


def _tc_red12(h, sel_prev, level):
    nb = h.shape[1]

    def body(*refs):
        if level == 0:
            h_ref, out = refs
            k_rem = T
            prev_key = jnp.int32(0)
            prev_above = jnp.int32(0)
        else:
            h_ref, sel_ref, out = refs
            prev_key = sel_ref[1]
            prev_above = sel_ref[2]
            k_rem = T - prev_above
        g = jnp.sum(h_ref[...], axis=0)
        b, cnt_above = _suffix_pick(g, k_rem)
        key = prev_key * 4096 + b
        above = prev_above + cnt_above
        i = lax.iota(jnp.int32, 128)
        out[...] = jnp.where(i == 1, key, jnp.where(i == 2, above, 0))

    args = (h,) if level == 0 else (h, sel_prev)
    return pl.pallas_call(
        body,
        out_shape=jax.ShapeDtypeStruct((128,), jnp.int32),
    )(*args)


def _tc_red3(h1, h2, h3, sel2):
    def _worker_sums(v):
        tj = lax.broadcasted_iota(jnp.int32, (W, W * 16), 1)
        wi = lax.broadcasted_iota(jnp.int32, (W, W * 16), 0)
        return jnp.sum(jnp.where(tj // 16 == wi, v[None, :], 0), axis=1)

    def body(h1r, h2r, h3r, selr, out):
        b12 = selr[1]
        above12 = selr[2]
        b1 = lax.shift_right_logical(b12, 12)
        b2 = b12 & 0xFFF
        g3 = jnp.sum(h3r[...], axis=0)
        k3 = T - above12
        b3, above3 = _suffix_pick(g3, k3)
        count_gt = above12 + above3
        n_eq = T - count_gt
        ustar = b12 * 256 + b3
        i1 = lax.iota(jnp.int32, NB1)
        i3 = lax.iota(jnp.int32, NB3)
        t1 = jnp.sum(jnp.where((i1 > b1)[None, :], h1r[...], 0), axis=1)
        t2 = jnp.sum(jnp.where((i1 > b2)[None, :], h2r[...], 0), axis=1)
        t3 = jnp.sum(jnp.where((i3 > b3)[None, :], h3r[...], 0), axis=1)
        te = jnp.sum(jnp.where((i3 == b3)[None, :], h3r[...], 0), axis=1)
        c_gt = _worker_sums(t1 + t2 + t3)
        c_eq = _worker_sums(te)
        wj = lax.broadcasted_iota(jnp.int32, (W, W), 1)
        wi = lax.broadcasted_iota(jnp.int32, (W, W), 0)
        excl = (wj < wi).astype(jnp.int32)
        eq_base = jnp.sum(excl * c_eq[None, :], axis=1)
        eq_keep = jnp.clip(n_eq - eq_base, 0, c_eq)
        cnt = c_gt + eq_keep
        keep_base = jnp.sum(excl * cnt[None, :], axis=1)
        out[...] = jnp.concatenate([
            jnp.zeros((1,), jnp.int32),
            jnp.stack([ustar, n_eq]).astype(jnp.int32),
            jnp.zeros((5,), jnp.int32),
            keep_base.astype(jnp.int32),
            eq_base.astype(jnp.int32),
            jnp.zeros((128 - 72,), jnp.int32),
        ])

    return pl.pallas_call(
        body,
        out_shape=jax.ShapeDtypeStruct((128,), jnp.int32),
    )(h1, h2, h3, sel2)


def _sc_write(norms, sel3):
    def body(norms_hbm, sel_hbm, keep_hbm, remap_hbm, nch, buf, remapb,
             idxl, selv, sem):
        wid = _wid()
        base_e = wid * PW
        pltpu.sync_copy(norms_hbm.at[pl.ds(base_e, PW)], nch)
        pltpu.sync_copy(sel_hbm, selv)
        ustar_s = _splat(selv, 1, 0)
        neq_s = _splat(selv, 2, 0)
        kb_s = _splat(selv, 8 + wid, 0)
        eqb_s = _splat(selv, 40 + wid, 0)
        iota = _iota16()

        def step(j, carry):
            krun, erun = carry
            u = nch[pl.ds(j * 16, 16)]
            gt = u > ustar_s
            eq = u == ustar_s
            eqc = plsc.cumsum(eq.astype(jnp.int32))
            eq_rank = eqb_s + erun + eqc
            keep = gt | (eq & (eq_rank <= neq_s))
            kc = plsc.cumsum(keep.astype(jnp.int32))
            pos_local = krun + kc - 1
            gidx = base_e + j * 16 + iota
            plsc.store_scatter(buf, [pos_local], gidx, mask=keep)
            remapb[pl.ds(j * 16, 16)] = jnp.where(keep, kb_s + pos_local, -1)
            krun = krun + jnp.sum(keep.astype(jnp.int32))
            erun = erun + jnp.sum(eq.astype(jnp.int32))
            return krun, erun

        cnt, _ = lax.fori_loop(0, PW // 16, step, (jnp.int32(0), jnp.int32(0)))
        last_s = plsc.load_gather(
            buf, [jnp.zeros((16,), jnp.int32) + jnp.maximum(cnt - 1, 0)])

        def pad(j, _):
            jv = j * 16 + iota
            m = jv < cnt
            plsc.store_scatter(buf, [jv], last_s, mask=~m)
            idxl[pl.ds(j * 16, 16)] = kb_s + jnp.maximum(
                jnp.minimum(jv, cnt - 1), 0)
            return 0

        lax.fori_loop(0, PW // 16, pad, 0)
        pltpu.sync_copy(remapb, remap_hbm.at[pl.ds(base_e, PW)])

        @pl.when(cnt > 0)
        def _():
            pltpu.async_copy(buf, keep_hbm.at[idxl], sem).wait()

    kfn = pl.kernel(
        body,
        out_type=(jax.ShapeDtypeStruct((T,), jnp.int32),
                  jax.ShapeDtypeStruct((E,), jnp.int32)),
        mesh=_mesh(),
        compiler_params=pltpu.CompilerParams(needs_layout_passes=False),
        scratch_types=[
            pltpu.VMEM((PW,), jnp.int32),
            pltpu.VMEM((PW,), jnp.int32),
            pltpu.VMEM((PW,), jnp.int32),
            pltpu.VMEM((PW,), jnp.int32),
            pltpu.VMEM((128,), jnp.int32),
            pltpu.SemaphoreType.DMA,
        ],
    )
    return kfn(norms, sel3)


def _sc_final(keep, remap, x, nbflat):
    XCH = 200
    NCH = SEG // XCH

    def body(keep_hbm, remap_hbm, x_hbm, nbf_hbm, xp_hbm, nbp_hbm,
             idxs, nbi, rv, outb, rowbuf, sem):
        wid = _wid()
        base_t = wid * SEG
        iota = _iota16()
        zeros = jnp.zeros((16,), jnp.int32)
        pltpu.sync_copy(keep_hbm.at[pl.ds(base_t, SEG)], idxs.at[pl.ds(0, SEG)])
        plsc.store_scatter(idxs, [SEG + iota], zeros, mask=iota < 8)

        def bld(j, _):
            kv = idxs[pl.ds(j * 16, 16)]
            for c in range(4):
                nbi[pl.ds(c * SEGP + j * 16, 16)] = kv * 4 + c
            return 0

        lax.fori_loop(0, SEGP // 16, bld, 0)
        pltpu.async_copy(nbf_hbm.at[nbi], rv, sem).wait()
        pltpu.async_copy(remap_hbm.at[rv], nbi, sem).wait()

        def fin(j, _):
            jv = j * 16 + iota
            m = jv < SEG
            for c in range(4):
                r = nbi[pl.ds(c * SEGP + j * 16, 16)]
                val = jnp.where(r < 0, base_t + jv, r)
                plsc.store_scatter(outb, [jv * 4 + c], val, mask=m)
            return 0

        lax.fori_loop(0, SEGP // 16, fin, 0)
        pltpu.sync_copy(outb, nbp_hbm.at[pl.ds(base_t * 4, SEG * 4)])

        def xch(cix, _):
            pltpu.async_copy(
                x_hbm.at[idxs.at[pl.ds(cix * XCH, XCH)]], rowbuf, sem).wait()
            pltpu.sync_copy(rowbuf, xp_hbm.at[pl.ds(base_t + cix * XCH, XCH)])
            return 0

        lax.fori_loop(0, NCH, xch, 0)

    kfn = pl.kernel(
        body,
        out_type=(jax.ShapeDtypeStruct((T, C), jnp.float32),
                  jax.ShapeDtypeStruct((T * 4,), jnp.int32)),
        mesh=_mesh(),
        compiler_params=pltpu.CompilerParams(needs_layout_passes=False),
        scratch_types=[
            pltpu.VMEM((SEGP,), jnp.int32),
            pltpu.VMEM((4 * SEGP,), jnp.int32),
            pltpu.VMEM((4 * SEGP,), jnp.int32),
            pltpu.VMEM((SEG * 4,), jnp.int32),
            pltpu.VMEM((XCH, C), jnp.float32),
            pltpu.SemaphoreType.DMA,
        ],
    )
    return kfn(keep, remap, x, nbflat)


def kernel(x, nb):
    norms = _norms(x)
    normsi = lax.bitcast_convert_type(norms, jnp.int32)
    h1 = _sc_hist(normsi, None, 0)
    sel1 = _tc_red12(h1.reshape(W * 16, NB1), None, 0)
    h2 = _sc_hist(normsi, sel1, 1)
    sel2 = _tc_red12(h2.reshape(W * 16, NB2), sel1, 1)
    h3 = _sc_hist(normsi, sel2, 2)
    sel3 = _tc_red3(h1.reshape(W * 16, NB1), h2.reshape(W * 16, NB2),
                    h3.reshape(W * 16, NB3), sel2)
    keep, remap = _sc_write(normsi, sel3)
    x_pool, nb_pool = _sc_final(keep, remap, x, nb.reshape(E * 4))
    return x_pool, nb_pool.reshape(T, 4), keep


# speedup vs baseline: 5.5872x; 2.2331x over previous
import jax
import jax.numpy as jnp
from jax import lax
from jax.experimental import pallas as pl
from jax.experimental.pallas import tpu as pltpu
from jax.experimental.pallas import tpu_sc as plsc

E = 320000
C = 128
T = 160000
NC = 2
NS = 16
W = NC * NS
PW = E // W
SEG = T // W
SEGP = SEG + 8
NB1 = 4096
NB2 = 4096
NB3 = 256
BLK = 512

_mesh = lambda: plsc.VectorSubcoreMesh(core_axis_name="c", subcore_axis_name="s")


def _wid():
    return lax.axis_index("s") * NC + lax.axis_index("c")


def _iota16():
    return lax.iota(jnp.int32, 16)


def _splat(vref, i, tz):
    return plsc.load_gather(vref, [jnp.zeros((16,), jnp.int32) + tz + i])


def _norms(x):
    def body(xr, out):
        xb = xr[...]
        y = xb * xb
        acc = y[:, 0:8]
        for k in range(1, 16):
            acc = acc + y[:, 8 * k:8 * k + 8]
        w = 8
        while w > 1:
            w //= 2
            acc = acc[:, :w] + acc[:, w:2 * w]
        out[...] = jnp.sqrt(acc[:, 0])

    return pl.pallas_call(
        body,
        grid=(E // BLK,),
        in_specs=[pl.BlockSpec((BLK, C), lambda i: (i, 0))],
        out_specs=pl.BlockSpec((BLK,), lambda i: (i,)),
        out_shape=jax.ShapeDtypeStruct((E,), jnp.float32),
    )(x)


def _sc_hist(norms, sel, level):
    nb = (NB1, NB2, NB3)[level]
    nsel = 0 if level == 0 else 128

    def body(*refs):
        if level == 0:
            norms_hbm, h_hbm, nch, hist, sem = refs
            selv = None
        else:
            norms_hbm, sel_hbm, h_hbm, nch, hist, selv, sem = refs
        wid = _wid()
        base = wid * PW
        pltpu.sync_copy(norms_hbm.at[pl.ds(base, PW)], nch)
        if level > 0:
            pltpu.sync_copy(sel_hbm, selv)
            key_s = _splat(selv, 1, 0)
        iota = _iota16()
        ones = jnp.ones((16,), jnp.int32)

        def clr(i, _):
            hist[pl.ds(i * 16, 16)] = jnp.zeros((16,), jnp.int32)
            return 0

        lax.fori_loop(0, nb, clr, 0)

        def acc(j, _):
            u = nch[pl.ds(j * 16, 16)]
            if level == 0:
                b = lax.shift_right_logical(u, 20)
                m = iota < 16
            elif level == 1:
                b = lax.shift_right_logical(u, 8) & 0xFFF
                m = lax.shift_right_logical(u, 20) == key_s
            else:
                b = u & 0xFF
                m = lax.shift_right_logical(u, 8) == key_s
            idx = iota * nb + b
            plsc.addupdate_scatter(hist, [idx], ones, mask=m)
            return 0

        lax.fori_loop(0, PW // 16, acc, 0)
        pltpu.sync_copy(hist, h_hbm.at[wid])

    scratch = [pltpu.VMEM((PW,), jnp.int32), pltpu.VMEM((16 * nb,), jnp.int32)]
    if level > 0:
        scratch.append(pltpu.VMEM((nsel,), jnp.int32))
    scratch.append(pltpu.SemaphoreType.DMA)

    kfn = pl.kernel(
        body,
        out_type=jax.ShapeDtypeStruct((W, 16 * nb), jnp.int32),
        mesh=_mesh(),
        compiler_params=pltpu.CompilerParams(needs_layout_passes=False),
        scratch_types=scratch,
    )
    return kfn(norms) if level == 0 else kfn(norms, sel)


def _suffix_pick(counts, kk):
    nb = counts.shape[0]
    rows, cols = nb // 128, 128
    M = counts.reshape(rows, cols)
    rs = jnp.sum(M, axis=1)
    rj = lax.broadcasted_iota(jnp.int32, (rows, rows), 1)
    ri = lax.broadcasted_iota(jnp.int32, (rows, rows), 0)
    S = jnp.sum(jnp.where(rj >= ri, rs[None, :], 0), axis=1)
    rstar = jnp.sum((S >= kk).astype(jnp.int32)) - 1
    riota = lax.broadcasted_iota(jnp.int32, (rows, cols), 0)
    row = jnp.sum(jnp.where(riota == rstar, M, 0), axis=0)
    ir = lax.iota(jnp.int32, rows)
    sx_star = jnp.sum(jnp.where(ir == rstar, S - rs, 0))
    cj = lax.broadcasted_iota(jnp.int32, (cols, cols), 1)
    ci = lax.broadcasted_iota(jnp.int32, (cols, cols), 0)
    csfx = jnp.sum(jnp.where(cj >= ci, row[None, :], 0), axis=1) + sx_star
    cstar = jnp.sum((csfx >= kk).astype(jnp.int32)) - 1
    b = rstar * cols + cstar
    ic = lax.iota(jnp.int32, cols)
    cnt_above = jnp.sum(jnp.where(ic == cstar, csfx - row, 0))
    return b, cnt_above


def _tc_red12(h, sel_prev, level):
    nb = h.shape[1]

    def body(*refs):
        if level == 0:
            h_ref, out = refs
            k_rem = T
            prev_key = jnp.int32(0)
            prev_above = jnp.int32(0)
        else:
            h_ref, sel_ref, out = refs
            prev_key = sel_ref[1]
            prev_above = sel_ref[2]
            k_rem = T - prev_above
        g = jnp.sum(h_ref[...], axis=0)
        b, cnt_above = _suffix_pick(g, k_rem)
        key = prev_key * 4096 + b
        above = prev_above + cnt_above
        i = lax.iota(jnp.int32, 128)
        out[...] = jnp.where(i == 1, key, jnp.where(i == 2, above, 0))

    args = (h,) if level == 0 else (h, sel_prev)
    return pl.pallas_call(
        body,
        out_shape=jax.ShapeDtypeStruct((128,), jnp.int32),
    )(*args)


def _tc_red3(h1, h2, h3, sel2):
    def _worker_sums(v):
        tj = lax.broadcasted_iota(jnp.int32, (W, W * 16), 1)
        wi = lax.broadcasted_iota(jnp.int32, (W, W * 16), 0)
        return jnp.sum(jnp.where(tj // 16 == wi, v[None, :], 0), axis=1)

    def body(h1r, h2r, h3r, selr, out):
        b12 = selr[1]
        above12 = selr[2]
        b1 = lax.shift_right_logical(b12, 12)
        b2 = b12 & 0xFFF
        g3 = jnp.sum(h3r[...], axis=0)
        k3 = T - above12
        b3, above3 = _suffix_pick(g3, k3)
        count_gt = above12 + above3
        n_eq = T - count_gt
        ustar = b12 * 256 + b3
        i1 = lax.iota(jnp.int32, NB1)
        i3 = lax.iota(jnp.int32, NB3)
        t1 = jnp.sum(jnp.where((i1 > b1)[None, :], h1r[...], 0), axis=1)
        t2 = jnp.sum(jnp.where((i1 > b2)[None, :], h2r[...], 0), axis=1)
        t3 = jnp.sum(jnp.where((i3 > b3)[None, :], h3r[...], 0), axis=1)
        te = jnp.sum(jnp.where((i3 == b3)[None, :], h3r[...], 0), axis=1)
        c_gt = _worker_sums(t1 + t2 + t3)
        c_eq = _worker_sums(te)
        wj = lax.broadcasted_iota(jnp.int32, (W, W), 1)
        wi = lax.broadcasted_iota(jnp.int32, (W, W), 0)
        excl = (wj < wi).astype(jnp.int32)
        eq_base = jnp.sum(excl * c_eq[None, :], axis=1)
        eq_keep = jnp.clip(n_eq - eq_base, 0, c_eq)
        cnt = c_gt + eq_keep
        keep_base = jnp.sum(excl * cnt[None, :], axis=1)
        out[...] = jnp.concatenate([
            jnp.zeros((1,), jnp.int32),
            jnp.stack([ustar, n_eq]).astype(jnp.int32),
            jnp.zeros((5,), jnp.int32),
            keep_base.astype(jnp.int32),
            eq_base.astype(jnp.int32),
            jnp.zeros((128 - 72,), jnp.int32),
        ])

    return pl.pallas_call(
        body,
        out_shape=jax.ShapeDtypeStruct((128,), jnp.int32),
    )(h1, h2, h3, sel2)


def _sc_write(norms, sel3):
    def body(norms_hbm, sel_hbm, keep_hbm, remap_hbm, nch, buf, remapb,
             selv, sem):
        wid = _wid()
        base_e = wid * PW
        pltpu.sync_copy(norms_hbm.at[pl.ds(base_e, PW)], nch)
        pltpu.sync_copy(sel_hbm, selv)
        ustar_s = _splat(selv, 1, 0)
        neq_s = _splat(selv, 2, 0)
        kb_s = _splat(selv, 8 + wid, 0)
        eqb_s = _splat(selv, 40 + wid, 0)
        iota = _iota16()

        def step(j, carry):
            krun, erun = carry
            u = nch[pl.ds(j * 16, 16)]
            gt = u > ustar_s
            eq = u == ustar_s
            eqc = plsc.cumsum(eq.astype(jnp.int32))
            eq_rank = eqb_s + erun + eqc
            keep = gt | (eq & (eq_rank <= neq_s))
            kc = plsc.cumsum(keep.astype(jnp.int32))
            pos_local = krun + kc - 1
            gidx = base_e + j * 16 + iota
            plsc.store_scatter(buf, [pos_local], gidx, mask=keep)
            remapb[pl.ds(j * 16, 16)] = jnp.where(keep, kb_s + pos_local, -1)
            krun = krun + jnp.sum(keep.astype(jnp.int32))
            erun = erun + jnp.sum(eq.astype(jnp.int32))
            return krun, erun

        lax.fori_loop(0, PW // 16, step, (jnp.int32(0), jnp.int32(0)))
        pltpu.sync_copy(remapb, remap_hbm.at[pl.ds(base_e, PW)])
        pltpu.sync_copy(buf, keep_hbm.at[wid])

    kfn = pl.kernel(
        body,
        out_type=(jax.ShapeDtypeStruct((W, PW), jnp.int32),
                  jax.ShapeDtypeStruct((E,), jnp.int32)),
        mesh=_mesh(),
        compiler_params=pltpu.CompilerParams(needs_layout_passes=False),
        scratch_types=[
            pltpu.VMEM((PW,), jnp.int32),
            pltpu.VMEM((PW,), jnp.int32),
            pltpu.VMEM((PW,), jnp.int32),
            pltpu.VMEM((128,), jnp.int32),
            pltpu.SemaphoreType.DMA,
        ],
    )
    return kfn(norms, sel3)


def _sc_final(kout, sel3, remap, x, nbflat):
    XCH = 200
    NCH = SEG // XCH

    def body(kout_hbm, sel_hbm, remap_hbm, x_hbm, nbf_hbm,
             keep_hbm, xp_hbm, nbp_hbm,
             idxs, nbi, rv, outb, rowbuf, selv, dwork, sem):
        wid = _wid()
        base_t = wid * SEG
        iota = _iota16()
        zeros = jnp.zeros((16,), jnp.int32)
        pltpu.sync_copy(sel_hbm, selv)

        def zero(j, _):
            dwork[pl.ds(j * 16, 16)] = zeros
            return 0

        lax.fori_loop(0, SEGP // 16, zero, 0)
        kb1 = selv[pl.ds(8, 16)]
        kb2 = selv[pl.ds(24, 16)]
        ones = jnp.ones((16,), jnp.int32)
        for kbv in (kb1, kb2):
            rel = kbv - base_t
            m = (rel >= 0) & (rel < SEG)
            plsc.addupdate_scatter(
                dwork, [jnp.clip(rel, 0, SEGP - 1)], ones, mask=m)
        base_strict = (jnp.sum((kb1 < base_t).astype(jnp.int32))
                       + jnp.sum((kb2 < base_t).astype(jnp.int32)))

        def srci(j, carry):
            run = carry
            d = dwork[pl.ds(j * 16, 16)]
            cum = plsc.cumsum(d)
            sw = base_strict - 1 + run + cum
            pv = base_t + j * 16 + iota
            kb_at = plsc.load_gather(selv, [8 + sw])
            s = jnp.clip(sw * PW + (pv - kb_at), 0, W * PW - 1)
            dwork[pl.ds(j * 16, 16)] = s
            return run + jnp.sum(d)

        lax.fori_loop(0, SEGP // 16, srci, jnp.int32(0))
        pltpu.async_copy(kout_hbm.at[dwork], idxs, sem).wait()
        pltpu.sync_copy(idxs.at[pl.ds(0, SEG)], keep_hbm.at[pl.ds(base_t, SEG)])
        plsc.store_scatter(idxs, [SEG + iota], zeros, mask=iota < 8)

        def bld(j, _):
            kv = idxs[pl.ds(j * 16, 16)]
            for c in range(4):
                nbi[pl.ds(c * SEGP + j * 16, 16)] = kv * 4 + c
            return 0

        lax.fori_loop(0, SEGP // 16, bld, 0)
        pltpu.async_copy(nbf_hbm.at[nbi], rv, sem).wait()
        pltpu.async_copy(remap_hbm.at[rv], nbi, sem).wait()

        def fin(j, _):
            jv = j * 16 + iota
            m = jv < SEG
            for c in range(4):
                r = nbi[pl.ds(c * SEGP + j * 16, 16)]
                val = jnp.where(r < 0, base_t + jv, r)
                plsc.store_scatter(outb, [jv * 4 + c], val, mask=m)
            return 0

        lax.fori_loop(0, SEGP // 16, fin, 0)
        pltpu.sync_copy(outb, nbp_hbm.at[pl.ds(base_t * 4, SEG * 4)])

        def xch(cix, _):
            pltpu.async_copy(
                x_hbm.at[idxs.at[pl.ds(cix * XCH, XCH)]], rowbuf, sem).wait()
            pltpu.sync_copy(rowbuf, xp_hbm.at[pl.ds(base_t + cix * XCH, XCH)])
            return 0

        lax.fori_loop(0, NCH, xch, 0)

    kfn = pl.kernel(
        body,
        out_type=(jax.ShapeDtypeStruct((T,), jnp.int32),
                  jax.ShapeDtypeStruct((T, C), jnp.float32),
                  jax.ShapeDtypeStruct((T * 4,), jnp.int32)),
        mesh=_mesh(),
        compiler_params=pltpu.CompilerParams(needs_layout_passes=False),
        scratch_types=[
            pltpu.VMEM((SEGP,), jnp.int32),
            pltpu.VMEM((4 * SEGP,), jnp.int32),
            pltpu.VMEM((4 * SEGP,), jnp.int32),
            pltpu.VMEM((SEG * 4,), jnp.int32),
            pltpu.VMEM((XCH, C), jnp.float32),
            pltpu.VMEM((128,), jnp.int32),
            pltpu.VMEM((SEGP,), jnp.int32),
            pltpu.SemaphoreType.DMA,
        ],
    )
    return kfn(kout, sel3, remap, x, nbflat)


def kernel(x, nb):
    norms = _norms(x)
    normsi = lax.bitcast_convert_type(norms, jnp.int32)
    h1 = _sc_hist(normsi, None, 0)
    sel1 = _tc_red12(h1.reshape(W * 16, NB1), None, 0)
    h2 = _sc_hist(normsi, sel1, 1)
    sel2 = _tc_red12(h2.reshape(W * 16, NB2), sel1, 1)
    h3 = _sc_hist(normsi, sel2, 2)
    sel3 = _tc_red3(h1.reshape(W * 16, NB1), h2.reshape(W * 16, NB2),
                    h3.reshape(W * 16, NB3), sel2)
    kout, remap = _sc_write(normsi, sel3)
    keep, x_pool, nb_pool = _sc_final(kout.reshape(W * PW), sel3, remap, x,
                                      nb.reshape(E * 4))
    return x_pool, nb_pool.reshape(T, 4), keep
